# baseline (speedup 1.0000x reference)
import functools
import math

import jax
import jax.numpy as jnp
import numpy as np
from jax import lax
from jax.experimental import pallas as pl
from jax.experimental.pallas import tpu as pltpu
from jax.experimental.pallas import tpu_sc as plsc

_EMBED_DIM = 64
_BATCH = 4096
_SEQ = 200
_N = _BATCH * _SEQ

_NC = 2
_NS = 16
_NW = _NC * _NS

_BW = _BATCH // _NW
_TB = _EMBED_DIM // 8
_CB = _BATCH // 128


def _pe_table_np():
    pos = np.arange(_SEQ, dtype=np.float64)[:, None]
    i = np.arange(0, _EMBED_DIM, 2, dtype=np.float64)
    div = np.exp(-(math.log(10000.0) * i / _EMBED_DIM))
    pe = np.zeros((_SEQ, _EMBED_DIM), dtype=np.float32)
    pe[:, 0::2] = np.sin(pos * div).astype(np.float32)
    pe[:, 1::2] = np.cos(pos * div).astype(np.float32)
    return pe


_PE_NP = _pe_table_np()


def _emb_body(xt_hbm, table_hbm, pe_hbm, out_hbm,
              idx_all, rb0, rb1, rb2, rb3, tb0, tb1, pe_v,
              gs0, gs1, gs2, gs3, os0, os1):
    rb = [rb0, rb1, rb2, rb3]
    tb = [tb0, tb1]
    gsem = [gs0, gs1, gs2, gs3]
    osem = [os0, os1]

    wid = lax.axis_index("s") * _NC + lax.axis_index("c")

    pltpu.sync_copy(pe_hbm, pe_v)
    pltpu.sync_copy(xt_hbm.at[:, pl.ds(wid * _BW, _BW)], idx_all)

    iota = lax.iota(jnp.int32, 16)
    rowvs = [b0 + iota for b0 in range(0, _BW, 16)]

    def start_gather(s, p):
        pltpu.async_copy(table_hbm.at[idx_all.at[s]], rb[p], gsem[p])

    def wait_gather(s, p):
        pltpu.make_async_copy(table_hbm.at[idx_all.at[s]], rb[p], gsem[p]).wait()

    def out_copy(s, p, start):
        for t in range(_TB):
            cp = pltpu.make_async_copy(
                tb[p].at[pl.ds(t * 8, 8), :], out_hbm.at[s, t, wid], osem[p]
            )
            if start:
                cp.start()
            else:
                cp.wait()

    for p in range(4):
        start_gather(p, p)

    def task(t4, carry):
        for p in range(4):
            s = 4 * t4 + p
            q = p & 1
            wait_gather(s, p)

            @pl.when(s >= 2)
            def _():
                out_copy(s, q, start=False)

            s64 = s * _EMBED_DIM

            @plsc.parallel_loop(0, 16)
            def _(k):
                rot = (iota + k) & 15
                for d0 in range(0, _EMBED_DIM, 16):
                    colv = d0 + rot
                    pdv = plsc.load_gather(pe_v, [s64 + colv])
                    for rowv in rowvs:
                        v = plsc.load_gather(rb[p], [rowv, colv])
                        plsc.store_scatter(tb[q], [colv, rowv], v + pdv)

            out_copy(s, q, start=True)

            @pl.when(s + 4 < _SEQ)
            def _():
                start_gather(s + 4, p)
        return carry

    lax.fori_loop(0, _SEQ // 4, task, 0)

    for p in range(2):
        out_copy(_SEQ - 2 + p, p, start=False)


_emb = functools.partial(
    pl.kernel,
    out_type=jax.ShapeDtypeStruct((_SEQ, _TB, _CB, 8, 128), jnp.float32),
    mesh=plsc.VectorSubcoreMesh(core_axis_name="c", subcore_axis_name="s"),
    scratch_types=[
        pltpu.VMEM((_SEQ, _BW), jnp.int32),
        pltpu.VMEM((_BW, _EMBED_DIM), jnp.float32),
        pltpu.VMEM((_BW, _EMBED_DIM), jnp.float32),
        pltpu.VMEM((_BW, _EMBED_DIM), jnp.float32),
        pltpu.VMEM((_BW, _EMBED_DIM), jnp.float32),
        pltpu.VMEM((_EMBED_DIM, _BW), jnp.float32),
        pltpu.VMEM((_EMBED_DIM, _BW), jnp.float32),
        pltpu.VMEM((_SEQ * _EMBED_DIM,), jnp.float32),
        pltpu.SemaphoreType.DMA,
        pltpu.SemaphoreType.DMA,
        pltpu.SemaphoreType.DMA,
        pltpu.SemaphoreType.DMA,
        pltpu.SemaphoreType.DMA,
        pltpu.SemaphoreType.DMA,
    ],
    compiler_params=pltpu.CompilerParams(
        use_tc_tiling_on_sc=False, needs_layout_passes=False
    ),
)(_emb_body)


@jax.jit
def _run(xt, table, pe):
    y5 = _emb(xt, table, pe)
    return y5.transpose(2, 4, 0, 1, 3).reshape(_BATCH, _SEQ, _EMBED_DIM)


def kernel(x, token_embedding_weight):
    xt = jnp.swapaxes(x, 0, 1).astype(jnp.int32)
    pe = jnp.asarray(_PE_NP.reshape(-1))
    return _run(xt, token_embedding_weight, pe)


# speedup vs baseline: 1.0325x; 1.0325x over previous
import functools
import math

import jax
import jax.numpy as jnp
import numpy as np
from jax import lax
from jax.experimental import pallas as pl
from jax.experimental.pallas import tpu as pltpu
from jax.experimental.pallas import tpu_sc as plsc

_EMBED_DIM = 64
_BATCH = 4096
_SEQ = 200
_N = _BATCH * _SEQ

_NC = 2
_NS = 16
_NW = _NC * _NS

_BW = _BATCH // _NW
_TB = _EMBED_DIM // 8
_CB = _BATCH // 128


def _pe_table_np():
    pos = np.arange(_SEQ, dtype=np.float64)[:, None]
    i = np.arange(0, _EMBED_DIM, 2, dtype=np.float64)
    div = np.exp(-(math.log(10000.0) * i / _EMBED_DIM))
    pe = np.zeros((_SEQ, _EMBED_DIM), dtype=np.float32)
    pe[:, 0::2] = np.sin(pos * div).astype(np.float32)
    pe[:, 1::2] = np.cos(pos * div).astype(np.float32)
    return pe


_PE_NP = _pe_table_np()


def _emb_body(xt_hbm, table_hbm, pe_hbm, out_hbm,
              idx_all, rb0, rb1, tb0, tb1, pe_v, gs0, gs1, os0, os1):
    rb = [rb0, rb1]
    tb = [tb0, tb1]
    gsem = [gs0, gs1]
    osem = [os0, os1]

    wid = lax.axis_index("s") * _NC + lax.axis_index("c")

    pltpu.sync_copy(pe_hbm, pe_v)
    pltpu.sync_copy(xt_hbm.at[:, pl.ds(wid * _BW, _BW)], idx_all)

    iota = lax.iota(jnp.int32, 16)
    rowvs = [b0 + iota for b0 in range(0, _BW, 16)]

    def start_gather(s, p):
        pltpu.async_copy(table_hbm.at[idx_all.at[s]], rb[p], gsem[p])

    def wait_gather(s, p):
        pltpu.make_async_copy(table_hbm.at[idx_all.at[s]], rb[p], gsem[p]).wait()

    def out_copy(s, p, start):
        cp = pltpu.make_async_copy(tb[p], out_hbm.at[s, :, wid], osem[p])
        if start:
            cp.start()
        else:
            cp.wait()

    for p in range(2):
        start_gather(p, p)

    def task(t2, carry):
        for p in range(2):
            s = 2 * t2 + p
            wait_gather(s, p)

            @pl.when(s >= 2)
            def _():
                out_copy(s, p, start=False)

            s64 = s * _EMBED_DIM

            @plsc.parallel_loop(0, 16)
            def _(k):
                rot = (iota + k) & 15
                for d0 in range(0, _EMBED_DIM, 16):
                    colv = d0 + rot
                    tv = colv >> 3
                    iv = colv & 7
                    pdv = plsc.load_gather(pe_v, [s64 + colv])
                    for rowv in rowvs:
                        v = plsc.load_gather(rb[p], [rowv, colv])
                        plsc.store_scatter(tb[p], [tv, iv, rowv], v + pdv)

            out_copy(s, p, start=True)

            @pl.when(s + 2 < _SEQ)
            def _():
                start_gather(s + 2, p)
        return carry

    lax.fori_loop(0, _SEQ // 2, task, 0)

    for p in range(2):
        out_copy(_SEQ - 2 + p, p, start=False)


_emb = functools.partial(
    pl.kernel,
    out_type=jax.ShapeDtypeStruct((_SEQ, _TB, _CB, 8, 128), jnp.float32),
    mesh=plsc.VectorSubcoreMesh(core_axis_name="c", subcore_axis_name="s"),
    scratch_types=[
        pltpu.VMEM((_SEQ, _BW), jnp.int32),
        pltpu.VMEM((_BW, _EMBED_DIM), jnp.float32),
        pltpu.VMEM((_BW, _EMBED_DIM), jnp.float32),
        pltpu.VMEM((_TB, 8, _BW), jnp.float32),
        pltpu.VMEM((_TB, 8, _BW), jnp.float32),
        pltpu.VMEM((_SEQ * _EMBED_DIM,), jnp.float32),
        pltpu.SemaphoreType.DMA,
        pltpu.SemaphoreType.DMA,
        pltpu.SemaphoreType.DMA,
        pltpu.SemaphoreType.DMA,
    ],
    compiler_params=pltpu.CompilerParams(
        use_tc_tiling_on_sc=False, needs_layout_passes=False
    ),
)(_emb_body)


@jax.jit
def _run(xt, table, pe):
    y5 = _emb(xt, table, pe)
    return y5.transpose(2, 4, 0, 1, 3).reshape(_BATCH, _SEQ, _EMBED_DIM)


def kernel(x, token_embedding_weight):
    xt = jnp.swapaxes(x, 0, 1).astype(jnp.int32)
    pe = jnp.asarray(_PE_NP.reshape(-1))
    return _run(xt, token_embedding_weight, pe)
